# baseline (speedup 1.0000x reference)
import functools
import math

import jax
import jax.numpy as jnp
import numpy as np
from jax import lax
from jax.experimental import pallas as pl
from jax.experimental.pallas import tpu as pltpu
from jax.experimental.pallas import tpu_sc as plsc

_DIM = 1024
_HALF = _DIM // 2
_BASE = 10000.0
_DIST_SCALE = 16.0
_INV_SQRT_DIM = 1.0 / math.sqrt(_DIM)


def _fit_turn_polys():
    v = np.linspace(-0.5, 0.5, 40001)
    w = v * v
    cosy = np.cos(2 * np.pi * v)
    siny = np.where(v == 0, 2 * np.pi, np.sin(2 * np.pi * v) / np.where(v == 0, 1, v))
    pc = np.polynomial.Polynomial.fit(w, cosy, 3).convert().coef
    ps = np.polynomial.Polynomial.fit(w, siny, 3).convert().coef
    return (tuple(float(c) * _INV_SQRT_DIM for c in pc),
            tuple(float(c) * _INV_SQRT_DIM for c in ps))


_COS_COEF, _SIN_COEF = _fit_turn_polys()


def _freq_reduction_consts():
    inv_freq32 = (
        1.0 / (_BASE ** (np.arange(0, _DIM, 2).astype(np.float32) / np.float32(_DIM)))
    ).astype(np.float32)
    f = np.mod(_DIST_SCALE * inv_freq32.astype(np.float64) / (2 * np.pi), 1.0)
    return f.astype(np.float32).reshape(1, _HALF)


_F_TURNS = _freq_reduction_consts()


def _sc_gather(weight, idx_flat, n_tokens):
    info = plsc.get_sparse_core_info()
    nw = info.num_cores * info.num_subcores
    b_per_w = n_tokens // nw
    chunk = 32
    n_chunks = b_per_w // chunk
    mesh = plsc.VectorSubcoreMesh(core_axis_name="c", subcore_axis_name="s")

    @functools.partial(
        pl.kernel,
        mesh=mesh,
        out_type=jax.ShapeDtypeStruct((n_tokens, _DIM), jnp.float32),
        scratch_types=[
            pltpu.VMEM((b_per_w,), jnp.int32),
            pltpu.VMEM((chunk, _DIM), jnp.float32),
            pltpu.VMEM((chunk, _DIM), jnp.float32),
            pltpu.VMEM((chunk, _DIM), jnp.float32),
            pltpu.SemaphoreType.DMA,
            pltpu.SemaphoreType.DMA,
            pltpu.SemaphoreType.DMA,
            pltpu.SemaphoreType.DMA,
            pltpu.SemaphoreType.DMA,
            pltpu.SemaphoreType.DMA,
        ],
    )
    def gather_kernel(table_hbm, idx_hbm, out_hbm, idx_v,
                      rows0, rows1, rows2, sg0, sg1, sg2, ss0, ss1, ss2):
        wid = lax.axis_index("s") * info.num_cores + lax.axis_index("c")
        base = wid * b_per_w
        pltpu.sync_copy(idx_hbm.at[pl.ds(base, b_per_w)], idx_v)

        bufs = (rows0, rows1, rows2)
        gsems = (sg0, sg1, sg2)
        ssems = (ss0, ss1, ss2)
        nbuf = 3

        def gather_start(j):
            pltpu.async_copy(
                table_hbm.at[idx_v.at[pl.ds(j * chunk, chunk)]],
                bufs[j % nbuf], gsems[j % nbuf],
            )

        def gather_wait(j):
            pltpu.make_async_copy(
                table_hbm.at[idx_v.at[pl.ds(j * chunk, chunk)]],
                bufs[j % nbuf], gsems[j % nbuf],
            ).wait()

        def scatter_start(j):
            pltpu.async_copy(
                bufs[j % nbuf], out_hbm.at[pl.ds(base + j * chunk, chunk)],
                ssems[j % nbuf],
            )

        def scatter_wait(j):
            pltpu.make_async_copy(
                bufs[j % nbuf], out_hbm.at[pl.ds(base + j * chunk, chunk)],
                ssems[j % nbuf],
            ).wait()

        gather_start(0)
        gather_start(1)
        for j in range(n_chunks):
            gather_wait(j)
            if j + 2 < n_chunks:
                if j >= 1:
                    scatter_wait(j - 1)
                gather_start(j + 2)
            scatter_start(j)
        for j in range(n_chunks - 3, n_chunks):
            scatter_wait(j)

    return gather_kernel(weight, idx_flat)


def _horner(w, coef):
    acc = jnp.full_like(w, coef[-1])
    for c in coef[-2::-1]:
        acc = acc * w + c
    return acc


def _rotary_body(f_ref, n_ref, x_ref, o_ref):
    n = n_ref[...]
    f = f_ref[...]
    z = n * f
    v = z - jnp.round(z)
    w = v * v
    c = _horner(w, _COS_COEF)
    s = v * _horner(w, _SIN_COEF)
    x1 = x_ref[:, :_HALF]
    x2 = x_ref[:, _HALF:]
    o_ref[:, :_HALF] = x1 * c - x2 * s
    o_ref[:, _HALF:] = x2 * c + x1 * s


_FUSE_G = 8


def _fused_body(ids_ref, f_ref, n_ref, *refs):
    o_ref = refs[-1]
    x = jnp.concatenate([r[...] for r in refs[:-1]], axis=0)[:, 0, :]
    n = n_ref[...]
    f = f_ref[...]
    z = n * f
    v = z - jnp.round(z)
    w = v * v
    c = _horner(w, _COS_COEF)
    s = v * _horner(w, _SIN_COEF)
    x1 = x[:, :_HALF]
    x2 = x[:, _HALF:]
    o_ref[:, :_HALF] = x1 * c - x2 * s
    o_ref[:, _HALF:] = x2 * c + x1 * s


def _fused_gather_rotary_tc(weight, idx_lo, nsub, n_total, n_lo):
    g = _FUSE_G
    grid = (n_lo // g,)

    def w_spec(k):
        return pl.BlockSpec((1, 1, _DIM), lambda i, s, k=k: (s[g * i + k], 0, 0))

    grid_spec = pltpu.PrefetchScalarGridSpec(
        num_scalar_prefetch=1,
        grid=grid,
        in_specs=[
            pl.BlockSpec((1, _HALF), lambda i, s: (0, 0)),
            pl.BlockSpec((g, 1), lambda i, s: (i, 0)),
            *[w_spec(k) for k in range(g)],
        ],
        out_specs=pl.BlockSpec((g, _DIM), lambda i, s: (i, 0)),
    )
    return pl.pallas_call(
        _fused_body,
        grid_spec=grid_spec,
        out_shape=jax.ShapeDtypeStruct((n_total, _DIM), jnp.float32),
    )(idx_lo, jnp.asarray(_F_TURNS), nsub, *([weight[:, None, :]] * g))


def _rotary_body_aliased(f_ref, n_ref, x_ref, prev_ref, o_ref):
    del prev_ref
    _rotary_body(f_ref, n_ref, x_ref, o_ref)


def _rotary_tc_hi(embeds_hi, nsub, out_prev, n_total):
    n_hi = embeds_hi.shape[0]
    t = 512
    grid = (n_hi // t,)
    off = (n_total - n_hi) // t
    return pl.pallas_call(
        _rotary_body_aliased,
        grid=grid,
        in_specs=[
            pl.BlockSpec((1, _HALF), lambda i: (0, 0)),
            pl.BlockSpec((t, 1), lambda i: (off + i, 0)),
            pl.BlockSpec((t, _DIM), lambda i: (i, 0)),
            pl.BlockSpec((8, 128), lambda i: (0, 0)),
        ],
        out_specs=pl.BlockSpec((t, _DIM), lambda i: (off + i, 0)),
        out_shape=jax.ShapeDtypeStruct((n_total, _DIM), jnp.float32),
        input_output_aliases={3: 0},
    )(jnp.asarray(_F_TURNS), nsub, embeds_hi, out_prev)


def kernel(ids, ids_sub, weight):
    b, s = ids.shape
    n = b * s
    n_lo = n // 2
    idx = ids.reshape(n)
    nsub = ids_sub.astype(jnp.float32).reshape(n, 1)
    embeds_hi = _sc_gather(weight, lax.slice(idx, (n_lo,), (n,)), n - n_lo)
    out_lo = _fused_gather_rotary_tc(
        weight, lax.slice(idx, (0,), (n_lo,)), nsub, n, n_lo
    )
    out = _rotary_tc_hi(embeds_hi, nsub, out_lo, n)
    return out.reshape(b, s, _DIM)


# speedup vs baseline: 7.3380x; 7.3380x over previous
import functools
import math

import jax
import jax.numpy as jnp
import numpy as np
from jax import lax
from jax.experimental import pallas as pl
from jax.experimental.pallas import tpu as pltpu
from jax.experimental.pallas import tpu_sc as plsc

_DIM = 1024
_HALF = _DIM // 2
_BASE = 10000.0
_DIST_SCALE = 16.0
_INV_SQRT_DIM = 1.0 / math.sqrt(_DIM)


def _fit_turn_polys():
    v = np.linspace(-0.5, 0.5, 40001)
    w = v * v
    cosy = np.cos(2 * np.pi * v)
    siny = np.where(v == 0, 2 * np.pi, np.sin(2 * np.pi * v) / np.where(v == 0, 1, v))
    pc = np.polynomial.Polynomial.fit(w, cosy, 3).convert().coef
    ps = np.polynomial.Polynomial.fit(w, siny, 3).convert().coef
    return (tuple(float(c) * _INV_SQRT_DIM for c in pc),
            tuple(float(c) * _INV_SQRT_DIM for c in ps))


_COS_COEF, _SIN_COEF = _fit_turn_polys()


def _freq_reduction_consts():
    inv_freq32 = (
        1.0 / (_BASE ** (np.arange(0, _DIM, 2).astype(np.float32) / np.float32(_DIM)))
    ).astype(np.float32)
    f = np.mod(_DIST_SCALE * inv_freq32.astype(np.float64) / (2 * np.pi), 1.0)
    return f.astype(np.float32).reshape(1, _HALF)


_F_TURNS = _freq_reduction_consts()


def _sc_gather(weight, idx_flat, n_tokens):
    info = plsc.get_sparse_core_info()
    nw = info.num_cores * info.num_subcores
    b_per_w = n_tokens // nw
    chunk = 32
    n_chunks = b_per_w // chunk
    mesh = plsc.VectorSubcoreMesh(core_axis_name="c", subcore_axis_name="s")

    @functools.partial(
        pl.kernel,
        mesh=mesh,
        out_type=jax.ShapeDtypeStruct((n_tokens, _DIM), jnp.float32),
        scratch_types=[
            pltpu.VMEM((b_per_w,), jnp.int32),
            pltpu.VMEM((chunk, _DIM), jnp.float32),
            pltpu.VMEM((chunk, _DIM), jnp.float32),
            pltpu.VMEM((chunk, _DIM), jnp.float32),
            pltpu.SemaphoreType.DMA,
            pltpu.SemaphoreType.DMA,
            pltpu.SemaphoreType.DMA,
            pltpu.SemaphoreType.DMA,
            pltpu.SemaphoreType.DMA,
            pltpu.SemaphoreType.DMA,
        ],
    )
    def gather_kernel(table_hbm, idx_hbm, out_hbm, idx_v,
                      rows0, rows1, rows2, sg0, sg1, sg2, ss0, ss1, ss2):
        wid = lax.axis_index("s") * info.num_cores + lax.axis_index("c")
        base = wid * b_per_w
        pltpu.sync_copy(idx_hbm.at[pl.ds(base, b_per_w)], idx_v)

        bufs = (rows0, rows1, rows2)
        gsems = (sg0, sg1, sg2)
        ssems = (ss0, ss1, ss2)
        nbuf = 3

        def gather_start(j):
            pltpu.async_copy(
                table_hbm.at[idx_v.at[pl.ds(j * chunk, chunk)]],
                bufs[j % nbuf], gsems[j % nbuf],
            )

        def gather_wait(j):
            pltpu.make_async_copy(
                table_hbm.at[idx_v.at[pl.ds(j * chunk, chunk)]],
                bufs[j % nbuf], gsems[j % nbuf],
            ).wait()

        def scatter_start(j):
            pltpu.async_copy(
                bufs[j % nbuf], out_hbm.at[pl.ds(base + j * chunk, chunk)],
                ssems[j % nbuf],
            )

        def scatter_wait(j):
            pltpu.make_async_copy(
                bufs[j % nbuf], out_hbm.at[pl.ds(base + j * chunk, chunk)],
                ssems[j % nbuf],
            ).wait()

        gather_start(0)
        gather_start(1)
        for j in range(n_chunks):
            gather_wait(j)
            if j + 2 < n_chunks:
                if j >= 1:
                    scatter_wait(j - 1)
                gather_start(j + 2)
            scatter_start(j)
        for j in range(n_chunks - 3, n_chunks):
            scatter_wait(j)

    return gather_kernel(weight, idx_flat)


def _horner(w, coef):
    acc = jnp.full_like(w, coef[-1])
    for c in coef[-2::-1]:
        acc = acc * w + c
    return acc


def _rotary_body(f_ref, n_ref, x_ref, o_ref):
    n = n_ref[...]
    f = f_ref[...]
    z = n * f
    v = z - jnp.round(z)
    w = v * v
    c = _horner(w, _COS_COEF)
    s = v * _horner(w, _SIN_COEF)
    x1 = x_ref[:, :_HALF]
    x2 = x_ref[:, _HALF:]
    o_ref[:, :_HALF] = x1 * c - x2 * s
    o_ref[:, _HALF:] = x2 * c + x1 * s


def _rotary_tc(embeds, nsub):
    n_tokens = embeds.shape[0]
    t = 512
    grid = (n_tokens // t,)
    return pl.pallas_call(
        _rotary_body,
        grid=grid,
        in_specs=[
            pl.BlockSpec((1, _HALF), lambda i: (0, 0)),
            pl.BlockSpec((t, 1), lambda i: (i, 0)),
            pl.BlockSpec((t, _DIM), lambda i: (i, 0)),
        ],
        out_specs=pl.BlockSpec((t, _DIM), lambda i: (i, 0)),
        out_shape=jax.ShapeDtypeStruct((n_tokens, _DIM), jnp.float32),
    )(jnp.asarray(_F_TURNS), nsub, embeds)


def kernel(ids, ids_sub, weight):
    b, s = ids.shape
    n = b * s
    idx = ids.reshape(n)
    nsub = ids_sub.astype(jnp.float32).reshape(n, 1)
    embeds = _sc_gather(weight, idx, n)
    out = _rotary_tc(embeds, nsub)
    return out.reshape(b, s, _DIM)
